# baseline (speedup 1.0000x reference)
import functools

import jax
import jax.numpy as jnp
from jax.experimental import pallas as pl

_TOP_K = 2
_NUM_EXPERTS = 8
_BLOCK_T = 1024


def _router_block(x_ref, w_ref, idx_ref, gate_ref):
    logits = jax.lax.dot_general(
        x_ref[...],
        w_ref[...],
        dimension_numbers=(((1,), (0,)), ((), ())),
        preferred_element_type=jnp.float32,
    )
    col = jax.lax.broadcasted_iota(jnp.int32, logits.shape, 1)
    m1 = jnp.max(logits, axis=1, keepdims=True)
    i1 = jnp.min(jnp.where(logits == m1, col, _NUM_EXPERTS), axis=1, keepdims=True)
    masked = jnp.where(col == i1, -jnp.inf, logits)
    m2 = jnp.max(masked, axis=1, keepdims=True)
    i2 = jnp.min(jnp.where(masked == m2, col, _NUM_EXPERTS), axis=1, keepdims=True)
    e2 = jnp.exp(m2 - m1)
    denom = 1.0 + e2
    g1 = 1.0 / denom
    g2 = e2 / denom
    idx_ref[...] = jnp.concatenate([i1, i2], axis=1)
    gate_ref[...] = jnp.concatenate([g1, g2], axis=1)


@jax.jit
def kernel(x, W):
    n_tokens, d_model = x.shape
    grid = (n_tokens // _BLOCK_T,)
    wt = W.T
    idx, gates = pl.pallas_call(
        _router_block,
        grid=grid,
        in_specs=[
            pl.BlockSpec((_BLOCK_T, d_model), lambda i: (i, 0)),
            pl.BlockSpec((d_model, _NUM_EXPERTS), lambda i: (0, 0)),
        ],
        out_specs=[
            pl.BlockSpec((_BLOCK_T, _TOP_K), lambda i: (i, 0)),
            pl.BlockSpec((_BLOCK_T, _TOP_K), lambda i: (i, 0)),
        ],
        out_shape=[
            jax.ShapeDtypeStruct((n_tokens, _TOP_K), jnp.int32),
            jax.ShapeDtypeStruct((n_tokens, _TOP_K), jnp.float32),
        ],
    )(x, wt)
    return idx, gates


# speedup vs baseline: 1.9679x; 1.9679x over previous
import jax
import jax.numpy as jnp
from jax.experimental import pallas as pl

_TOP_K = 2
_NUM_EXPERTS = 8
_BLOCK_T = 2048


def _router_block(x_ref, w_ref, i1_ref, i2_ref, g1_ref, g2_ref):
    logits = jax.lax.dot_general(
        x_ref[...],
        w_ref[...],
        dimension_numbers=(((1,), (0,)), ((), ())),
        preferred_element_type=jnp.float32,
    )
    lt = logits.T
    e8 = jax.lax.broadcasted_iota(jnp.int32, lt.shape, 0)
    m1 = jnp.max(lt, axis=0, keepdims=True)
    i1 = jnp.min(jnp.where(lt == m1, e8, _NUM_EXPERTS), axis=0, keepdims=True)
    masked = jnp.where(e8 == i1, -jnp.inf, lt)
    m2 = jnp.max(masked, axis=0, keepdims=True)
    i2 = jnp.min(jnp.where(masked == m2, e8, _NUM_EXPERTS), axis=0, keepdims=True)
    ex = jnp.exp(m2 - m1)
    den = 1.0 + ex
    bt = lt.shape[1]
    i1_ref[...] = i1.reshape(1, 1, bt)
    i2_ref[...] = i2.reshape(1, 1, bt)
    g1_ref[...] = (1.0 / den).reshape(1, 1, bt)
    g2_ref[...] = (ex / den).reshape(1, 1, bt)


@jax.jit
def kernel(x, W):
    n_tokens, d_model = x.shape
    nb = n_tokens // _BLOCK_T
    wt = W.T
    row_spec = pl.BlockSpec((1, 1, _BLOCK_T), lambda i: (i, 0, 0))
    row_shape_i = jax.ShapeDtypeStruct((nb, 1, _BLOCK_T), jnp.int32)
    row_shape_f = jax.ShapeDtypeStruct((nb, 1, _BLOCK_T), jnp.float32)
    i1, i2, g1, g2 = pl.pallas_call(
        _router_block,
        grid=(nb,),
        in_specs=[
            pl.BlockSpec((_BLOCK_T, d_model), lambda i: (i, 0)),
            pl.BlockSpec((d_model, _NUM_EXPERTS), lambda i: (0, 0)),
        ],
        out_specs=[row_spec, row_spec, row_spec, row_spec],
        out_shape=[row_shape_i, row_shape_i, row_shape_f, row_shape_f],
    )(x, wt)
    idx = jnp.stack([i1.reshape(-1), i2.reshape(-1)], axis=1)
    gates = jnp.stack([g1.reshape(-1), g2.reshape(-1)], axis=1)
    return idx, gates


# speedup vs baseline: 2.0193x; 1.0261x over previous
import jax
import jax.numpy as jnp
from jax.experimental import pallas as pl

_TOP_K = 2
_NUM_EXPERTS = 8
_BLOCK_T = 4096


def _router_block(x_ref, w_ref, i1_ref, i2_ref, g1_ref, g2_ref):
    logits = jax.lax.dot_general(
        x_ref[...],
        w_ref[...],
        dimension_numbers=(((1,), (0,)), ((), ())),
        preferred_element_type=jnp.float32,
    )
    lt = logits.T
    e8 = jax.lax.broadcasted_iota(jnp.int32, lt.shape, 0)
    m1 = jnp.max(lt, axis=0, keepdims=True)
    i1 = jnp.min(jnp.where(lt == m1, e8, _NUM_EXPERTS), axis=0, keepdims=True)
    masked = jnp.where(e8 == i1, -jnp.inf, lt)
    m2 = jnp.max(masked, axis=0, keepdims=True)
    i2 = jnp.min(jnp.where(masked == m2, e8, _NUM_EXPERTS), axis=0, keepdims=True)
    ex = jnp.exp(m2 - m1)
    den = 1.0 + ex
    bt = lt.shape[1]
    i1_ref[...] = i1.reshape(1, 1, bt)
    i2_ref[...] = i2.reshape(1, 1, bt)
    g1_ref[...] = (1.0 / den).reshape(1, 1, bt)
    g2_ref[...] = (ex / den).reshape(1, 1, bt)


@jax.jit
def kernel(x, W):
    n_tokens, d_model = x.shape
    nb = n_tokens // _BLOCK_T
    wt = W.T
    row_spec = pl.BlockSpec((1, 1, _BLOCK_T), lambda i: (i, 0, 0))
    row_shape_i = jax.ShapeDtypeStruct((nb, 1, _BLOCK_T), jnp.int32)
    row_shape_f = jax.ShapeDtypeStruct((nb, 1, _BLOCK_T), jnp.float32)
    i1, i2, g1, g2 = pl.pallas_call(
        _router_block,
        grid=(nb,),
        in_specs=[
            pl.BlockSpec((_BLOCK_T, d_model), lambda i: (i, 0)),
            pl.BlockSpec((d_model, _NUM_EXPERTS), lambda i: (0, 0)),
        ],
        out_specs=[row_spec, row_spec, row_spec, row_spec],
        out_shape=[row_shape_i, row_shape_i, row_shape_f, row_shape_f],
    )(x, wt)
    idx = jnp.stack([i1.reshape(-1), i2.reshape(-1)], axis=1)
    gates = jnp.stack([g1.reshape(-1), g2.reshape(-1)], axis=1)
    return idx, gates
